# baseline (speedup 1.0000x reference)
import functools

import jax
import jax.numpy as jnp
from jax import lax
from jax.experimental import pallas as pl
from jax.experimental.pallas import tpu as pltpu
from jax.experimental.pallas import tpu_sc as plsc

D = 64
BLK = 128


def _sc_worker_count():
    try:
        info = plsc.get_sparse_core_info()
        return int(info.num_cores) * int(info.num_subcores), int(info.num_cores)
    except Exception:
        return 32, 2


@functools.lru_cache(maxsize=None)
def _build_sc_gather(B, L, S, NU, NI):
    NW, NC = _sc_worker_count()
    BIG = B * L
    PERW = BIG // NW
    CH = 80
    NCH = PERW // CH
    assert NCH * CH == PERW
    PERW_N = B // NW

    mesh = plsc.VectorSubcoreMesh(core_axis_name="c", subcore_axis_name="s")

    @functools.partial(
        pl.kernel,
        out_type=(
            jax.ShapeDtypeStruct((BIG, D), jnp.float32),
            jax.ShapeDtypeStruct((BIG, D), jnp.float32),
            jax.ShapeDtypeStruct((BIG, D), jnp.float32),
            jax.ShapeDtypeStruct((B, D), jnp.float32),
            jax.ShapeDtypeStruct((B, D), jnp.float32),
        ),
        mesh=mesh,
        compiler_params=pltpu.CompilerParams(use_tc_tiling_on_sc=False),
        scratch_types=[
            pltpu.VMEM((PERW,), jnp.int32),
            pltpu.VMEM((PERW, D), jnp.float32),
            pltpu.SemaphoreType.DMA,
        ],
    )
    def sc_gather(u2e, i2e, hu, hi, so, nu, ni,
                  e_iu, e_uu, e_soc, ru, ri, idx_v, rows_v, sem):
        wid = lax.axis_index("s") * NC + lax.axis_index("c")

        def gath(idx_hbm, table, out_hbm, count, nch, ch, base):
            pltpu.sync_copy(idx_hbm.at[pl.ds(base, count)],
                            idx_v.at[pl.ds(0, count)])

            def fire(c, carry):
                pltpu.async_copy(
                    table.at[idx_v.at[pl.ds(c * ch, ch)]],
                    rows_v.at[pl.ds(c * ch, ch)],
                    sem,
                )
                return carry

            lax.fori_loop(0, nch, fire, 0)
            pltpu.make_async_copy(out_hbm.at[pl.ds(base, count)],
                                  rows_v.at[pl.ds(0, count)], sem).wait()
            pltpu.sync_copy(rows_v.at[pl.ds(0, count)],
                            out_hbm.at[pl.ds(base, count)])

        base = wid * PERW
        gath(hu, i2e, e_iu, PERW, NCH, CH, base)
        gath(hi, u2e, e_uu, PERW, NCH, CH, base)
        gath(so, u2e, e_soc, PERW, NCH, CH, base)
        nbase = wid * PERW_N
        gath(nu, u2e, ru, PERW_N, 1, PERW_N, nbase)
        gath(ni, i2e, ri, PERW_N, 1, PERW_N, nbase)

    return sc_gather


(U_W1A, U_B1TAB, U_W2, U_A1A, U_A1B, U_A2, U_A3M, U_L1A, U_L1B,
 I_W1A, I_B1TAB, I_W2, I_A1A, I_A1B, I_A2, I_A3M, I_L1A, I_L1B,
 S_A1A, S_A1B, S_A2, S_A3M, S_L1A, S_L1B,
 H_WUA, H_WUB, H_WUR1, H_WUR2, H_WIR1, H_WIR2, H_WUI1A, H_WUI1B, H_WUI2) = \
    range(33)
NWS = 33

(BU_B2, BU_BA1, BU_BA2, BU_A3W, BU_BL1,
 BI_B2, BI_BA1, BI_BA2, BI_A3W, BI_BL1,
 BS_BA1, BS_BA2, BS_A3W, BS_BL1,
 BH_BWU, BH_BUR1, BH_BUR2, BH_BIR1, BH_BIR2, BH_BUI1, BH_BUI2, BH_W3) = \
    range(22)
NBS = 24


def _mm(x, w):
    return lax.dot_general(x, w, (((1,), (0,)), ((), ())),
                           preferred_element_type=jnp.float32)


def _tc_body(L, eiu_ref, euu_ref, esoc_ref, repu_ref, repi_ref,
             hur_ref, hir_ref, ws_ref, bs_ref, out_ref):
    relu = lambda x: jnp.maximum(x, 0.0)
    ws = lambda k: ws_ref[k]
    bs = lambda k: bs_ref[k][None, :]

    def attention_agg(o_flat, rep, a1a, a1b, ba1, a2, ba2, a3m):
        ra = _mm(rep, ws(a1b)) + bs(ba1)
        ra3 = lax.broadcast_in_dim(ra, (BLK, L, D), (0, 2))
        a = _mm(o_flat, ws(a1a)).reshape(BLK, L, D) + ra3
        a = relu(a).reshape(BLK * L, D)
        a = relu(_mm(a, ws(a2)) + bs(ba2))
        s3 = _mm(a, ws(a3m)).reshape(BLK, L, D)
        m = jnp.max(s3, axis=1, keepdims=True)
        e = jnp.exp(s3 - m)
        att = e / jnp.sum(e, axis=1, keepdims=True)
        return jnp.sum(o_flat.reshape(BLK, L, D) * att, axis=1)

    def one_hot64(idx_col):
        iota = lax.broadcasted_iota(jnp.int32, (BLK * L, D), 1)
        return (idx_col == iota).astype(jnp.float32)

    def ui_agg(e_ref, rep, hist_ref, w1a, b1tab, w2, b2,
               a1a, a1b, ba1, a2, ba2, a3m, l1a, l1b, bl1):
        oh = one_hot64(hist_ref[...])
        x = relu(_mm(e_ref[...], ws(w1a)) + _mm(oh, ws(b1tab)))
        o = relu(_mm(x, ws(w2)) + bs(b2))
        neigh = attention_agg(o, rep, a1a, a1b, ba1, a2, ba2, a3m)
        return relu(_mm(rep, ws(l1a)) + _mm(neigh, ws(l1b)) + bs(bl1))

    rep_u = repu_ref[...]
    rep_i = repi_ref[...]

    item_space = ui_agg(eiu_ref, rep_u, hur_ref,
                        U_W1A, U_B1TAB, U_W2, BU_B2,
                        U_A1A, U_A1B, BU_BA1, U_A2, BU_BA2, U_A3M,
                        U_L1A, U_L1B, BU_BL1)

    neigh_s = attention_agg(esoc_ref[...], rep_u,
                            S_A1A, S_A1B, BS_BA1, S_A2, BS_BA2, S_A3M)
    social_space = relu(_mm(rep_u, ws(S_L1A)) + _mm(neigh_s, ws(S_L1B))
                        + bs(BS_BL1))

    i_lat = ui_agg(euu_ref, rep_i, hir_ref,
                   I_W1A, I_B1TAB, I_W2, BI_B2,
                   I_A1A, I_A1B, BI_BA1, I_A2, BI_BA2, I_A3M,
                   I_L1A, I_L1B, BI_BL1)

    u_lat = relu(_mm(item_space, ws(H_WUA)) + _mm(social_space, ws(H_WUB))
                 + bs(BH_BWU))
    u_lat = relu(_mm(u_lat, ws(H_WUR1)) + bs(BH_BUR1))
    u_lat = _mm(u_lat, ws(H_WUR2)) + bs(BH_BUR2)
    i_lat = relu(_mm(i_lat, ws(H_WIR1)) + bs(BH_BIR1))
    i_lat = _mm(i_lat, ws(H_WIR2)) + bs(BH_BIR2)
    lat = relu(_mm(u_lat, ws(H_WUI1A)) + _mm(i_lat, ws(H_WUI1B)) + bs(BH_BUI1))
    lat = relu(_mm(lat, ws(H_WUI2)) + bs(BH_BUI2))
    score = jnp.sum(lat * bs_ref[BH_W3][None, :], axis=-1)
    out_ref[...] = score


def _tc_forward(L, e_iu, e_uu, e_soc, rep_u, rep_i, hur, hir, WS, BS,
                interpret=False):
    B = rep_u.shape[0]
    nblk = B // BLK
    grid = (nblk,)
    body = functools.partial(_tc_body, L)
    out = pl.pallas_call(
        body,
        grid=grid,
        in_specs=[
            pl.BlockSpec((BLK * L, D), lambda i: (i, 0)),
            pl.BlockSpec((BLK * L, D), lambda i: (i, 0)),
            pl.BlockSpec((BLK * L, D), lambda i: (i, 0)),
            pl.BlockSpec((BLK, D), lambda i: (i, 0)),
            pl.BlockSpec((BLK, D), lambda i: (i, 0)),
            pl.BlockSpec((BLK * L, 1), lambda i: (i, 0)),
            pl.BlockSpec((BLK * L, 1), lambda i: (i, 0)),
            pl.BlockSpec((NWS, D, D), lambda i: (0, 0, 0)),
            pl.BlockSpec((NBS, D), lambda i: (0, 0)),
        ],
        out_specs=pl.BlockSpec((BLK,), lambda i: (i,)),
        out_shape=jax.ShapeDtypeStruct((B,), jnp.float32),
        interpret=interpret,
    )(e_iu, e_uu, e_soc, rep_u, rep_i, hur, hir, WS, BS)
    return out


def _prep_weights(p):
    r2e = p['r2e']
    NR = r2e.shape[0]

    def tpose(l):
        return l['w'].T

    def pad_rows(m):
        return jnp.concatenate(
            [m, jnp.zeros((D - m.shape[0], D), jnp.float32)], axis=0)

    def a3mat(pa):
        return jnp.tile(pa['att3']['w'][0][:, None], (1, D))

    def agg_mats(pa):
        w1 = pa['w_r1']['w']
        b1tab = pad_rows(r2e @ w1[:, D:].T + pa['w_r1']['b'][None, :])
        return [w1[:, :D].T, b1tab, tpose(pa['w_r2']),
                pa['att1']['w'][:, :D].T, pa['att1']['w'][:, D:].T,
                tpose(pa['att2']), a3mat(pa),
                pa['linear1']['w'][:, :D].T, pa['linear1']['w'][:, D:].T]

    def agg_biases(pa):
        return [pa['w_r2']['b'], pa['att1']['b'], pa['att2']['b'],
                pa['att3']['w'][0], pa['linear1']['b']]

    def soc_mats(pa):
        return [pa['att1']['w'][:, :D].T, pa['att1']['w'][:, D:].T,
                tpose(pa['att2']), a3mat(pa),
                pa['linear1']['w'][:, :D].T, pa['linear1']['w'][:, D:].T]

    def soc_biases(pa):
        return [pa['att1']['b'], pa['att2']['b'],
                pa['att3']['w'][0], pa['linear1']['b']]

    g1, bb1 = p['bn1']['g'], p['bn1']['b']
    g2, bb2 = p['bn2']['g'], p['bn2']['b']
    g3, bb3 = p['bn3']['g'], p['bn3']['b']
    g4, bb4 = p['bn4']['g'], p['bn4']['b']

    wur1 = p['w_ur1']['w'].T * g1[None, :]
    bur1 = p['w_ur1']['b'] * g1 + bb1
    wir1 = p['w_ir1']['w'].T * g2[None, :]
    bir1 = p['w_ir1']['b'] * g2 + bb2
    wui1 = p['w_ui1']['w']
    wui1a = wui1[:, :D].T * g3[None, :]
    wui1b = wui1[:, D:].T * g3[None, :]
    bui1 = p['w_ui1']['b'] * g3 + bb3

    wui2 = p['w_ui2']['w'].T * g4[None, :]
    wui2 = jnp.concatenate(
        [wui2, jnp.zeros((D, D - 16), jnp.float32)], axis=1)
    bui2 = jnp.concatenate(
        [p['w_ui2']['b'] * g4 + bb4,
         jnp.ones((1,), jnp.float32),
         jnp.zeros((D - 17,), jnp.float32)])
    w3 = jnp.concatenate(
        [p['w_ui3']['w'][0], p['w_ui3']['b'],
         jnp.zeros((D - 17,), jnp.float32)])

    mats = (agg_mats(p['enc_u']) + agg_mats(p['enc_i']) + soc_mats(p['enc_s'])
            + [p['w_u']['w'][:, :D].T, p['w_u']['w'][:, D:].T,
               wur1, tpose(p['w_ur2']), wir1, tpose(p['w_ir2']),
               wui1a, wui1b, wui2])
    biases = (agg_biases(p['enc_u']) + agg_biases(p['enc_i'])
              + soc_biases(p['enc_s'])
              + [p['w_u']['b'], bur1, p['w_ur2']['b'], bir1, p['w_ir2']['b'],
                 bui1, bui2, w3]
              + [jnp.zeros((D,), jnp.float32)] * (NBS - 22))
    WS = jnp.stack(mats)
    BS = jnp.stack(biases)
    return WS, BS


def kernel(nodes_u, nodes_i, hist_u, hist_ur, hist_i, hist_ir, soc, params):
    p = params
    B, L = hist_u.shape
    S = soc.shape[1]
    NU = p['u2e'].shape[0]
    NI = p['i2e'].shape[0]

    WS, BS = _prep_weights(p)

    NCHUNK = 2
    Bc = B // NCHUNK
    sc_gather = _build_sc_gather(Bc, L, S, NU, NI)

    hu = hist_u.reshape(NCHUNK, Bc * L)
    hi = hist_i.reshape(NCHUNK, Bc * L)
    so = soc.reshape(NCHUNK, Bc * S)
    nu = nodes_u.reshape(NCHUNK, Bc)
    ni = nodes_i.reshape(NCHUNK, Bc)
    hur = hist_ur.reshape(NCHUNK, Bc * L, 1)
    hir = hist_ir.reshape(NCHUNK, Bc * L, 1)

    outs = []
    for c in range(NCHUNK):
        e_iu, e_uu, e_soc, rep_u, rep_i = sc_gather(
            p['u2e'], p['i2e'], hu[c], hi[c], so[c], nu[c], ni[c])
        outs.append(_tc_forward(L, e_iu, e_uu, e_soc, rep_u, rep_i,
                                hur[c], hir[c], WS, BS))
    return jnp.concatenate(outs)


# speedup vs baseline: 1.5071x; 1.5071x over previous
import functools

import jax
import jax.numpy as jnp
from jax import lax
from jax.experimental import pallas as pl
from jax.experimental.pallas import tpu as pltpu
from jax.experimental.pallas import tpu_sc as plsc

D = 64
BLK = 128
LP = 56


def _sc_worker_count():
    try:
        info = plsc.get_sparse_core_info()
        return int(info.num_cores) * int(info.num_subcores), int(info.num_cores)
    except Exception:
        return 32, 2


@functools.lru_cache(maxsize=None)
def _build_sc_gather(B, L, S, NU, NI):
    NW, NC = _sc_worker_count()
    BIG = B * L
    PERW = BIG // NW
    CH = 80
    NCH = PERW // CH
    assert NCH * CH == PERW
    PERW_B = B // NW
    PERW_N = B // NW

    mesh = plsc.VectorSubcoreMesh(core_axis_name="c", subcore_axis_name="s")

    @functools.partial(
        pl.kernel,
        out_type=(
            jax.ShapeDtypeStruct((B * LP, D), jnp.float32),
            jax.ShapeDtypeStruct((B * LP, D), jnp.float32),
            jax.ShapeDtypeStruct((B * LP, D), jnp.float32),
            jax.ShapeDtypeStruct((B, D), jnp.float32),
            jax.ShapeDtypeStruct((B, D), jnp.float32),
        ),
        mesh=mesh,
        compiler_params=pltpu.CompilerParams(use_tc_tiling_on_sc=False),
        scratch_types=[
            pltpu.VMEM((PERW,), jnp.int32),
            pltpu.VMEM((PERW, D), jnp.float32),
            pltpu.SemaphoreType.DMA,
        ],
    )
    def sc_gather(u2e, i2e, hu, hi, so, nu, ni,
                  e_iu, e_uu, e_soc, ru, ri, idx_v, rows_v, sem):
        wid = lax.axis_index("s") * NC + lax.axis_index("c")

        def gath(idx_hbm, table, out_hbm, count, nch, ch, base, out_base,
                 pad_out):
            pltpu.sync_copy(idx_hbm.at[pl.ds(base, count)],
                            idx_v.at[pl.ds(0, count)])

            def fire(c, carry):
                pltpu.async_copy(
                    table.at[idx_v.at[pl.ds(c * ch, ch)]],
                    rows_v.at[pl.ds(c * ch, ch)],
                    sem,
                )
                return carry

            lax.fori_loop(0, nch, fire, 0)
            pltpu.make_async_copy(out_hbm.at[pl.ds(out_base, count)],
                                  rows_v.at[pl.ds(0, count)], sem).wait()
            if not pad_out:
                pltpu.sync_copy(rows_v.at[pl.ds(0, count)],
                                out_hbm.at[pl.ds(out_base, count)])
            else:
                def wout(i, carry):
                    pltpu.async_copy(
                        rows_v.at[pl.ds(i * L, L)],
                        out_hbm.at[pl.ds(out_base + i * LP, L)],
                        sem,
                    )
                    return carry

                lax.fori_loop(0, PERW_B, wout, 0)
                pltpu.make_async_copy(out_hbm.at[pl.ds(out_base, count)],
                                      rows_v.at[pl.ds(0, count)], sem).wait()

        base = wid * PERW
        obase = wid * PERW_B * LP
        gath(hu, i2e, e_iu, PERW, NCH, CH, base, obase, True)
        gath(hi, u2e, e_uu, PERW, NCH, CH, base, obase, True)
        gath(so, u2e, e_soc, PERW, NCH, CH, base, obase, True)
        nbase = wid * PERW_N
        gath(nu, u2e, ru, PERW_N, 1, PERW_N, nbase, nbase, False)
        gath(ni, i2e, ri, PERW_N, 1, PERW_N, nbase, nbase, False)

    return sc_gather


(U_W1A, U_B1TAB, U_W2, U_A1A, U_A1B, U_A2, U_A3M, U_L1A, U_L1B,
 I_W1A, I_B1TAB, I_W2, I_A1A, I_A1B, I_A2, I_A3M, I_L1A, I_L1B,
 S_A1A, S_A1B, S_A2, S_A3M, S_L1A, S_L1B,
 H_WUA, H_WUB, H_WUR1, H_WUR2, H_WIR1, H_WIR2, H_WUI1A, H_WUI1B, H_WUI2) = \
    range(33)
NWS = 33

(BU_B2, BU_BA1, BU_BA2, BU_A3W, BU_BL1,
 BI_B2, BI_BA1, BI_BA2, BI_A3W, BI_BL1,
 BS_BA1, BS_BA2, BS_A3W, BS_BL1,
 BH_BWU, BH_BUR1, BH_BUR2, BH_BIR1, BH_BIR2, BH_BUI1, BH_BUI2, BH_W3) = \
    range(22)
NBS = 24


def _mm(x, w):
    return lax.dot_general(x, w, (((1,), (0,)), ((), ())),
                           preferred_element_type=jnp.float32)


def _tc_body(L, eiu_ref, euu_ref, esoc_ref, repu_ref, repi_ref,
             hur_ref, hir_ref, ws_ref, bs_ref, out_ref):
    relu = lambda x: jnp.maximum(x, 0.0)
    ws = lambda k: ws_ref[k]
    bs = lambda k: bs_ref[k][None, :]
    lmask = lax.broadcasted_iota(jnp.int32, (BLK, LP, 1), 1) < L

    def attention_agg(o_flat, rep, a1a, a1b, ba1, a2, ba2, a3m):
        ra = _mm(rep, ws(a1b)) + bs(ba1)
        ra3 = lax.broadcast_in_dim(ra, (BLK, LP, D), (0, 2))
        a = _mm(o_flat, ws(a1a)).reshape(BLK, LP, D) + ra3
        a = relu(a).reshape(BLK * LP, D)
        a = relu(_mm(a, ws(a2)) + bs(ba2))
        s3 = _mm(a, ws(a3m)).reshape(BLK, LP, D)
        s3 = jnp.where(lmask, s3, -1e30)
        m = jnp.max(s3, axis=1, keepdims=True)
        e = jnp.exp(s3 - m)
        att = e / jnp.sum(e, axis=1, keepdims=True)
        o3 = jnp.where(lmask, o_flat.reshape(BLK, LP, D), 0.0)
        return jnp.sum(o3 * att, axis=1)

    def one_hot64(idx_col):
        iota = lax.broadcasted_iota(jnp.int32, (BLK * LP, D), 1)
        return (idx_col == iota).astype(jnp.float32)

    def ui_agg(e_ref, rep, hist_ref, w1a, b1tab, w2, b2,
               a1a, a1b, ba1, a2, ba2, a3m, l1a, l1b, bl1):
        oh = one_hot64(hist_ref[...])
        x = relu(_mm(e_ref[...], ws(w1a)) + _mm(oh, ws(b1tab)))
        o = relu(_mm(x, ws(w2)) + bs(b2))
        neigh = attention_agg(o, rep, a1a, a1b, ba1, a2, ba2, a3m)
        return relu(_mm(rep, ws(l1a)) + _mm(neigh, ws(l1b)) + bs(bl1))

    rep_u = repu_ref[...]
    rep_i = repi_ref[...]

    item_space = ui_agg(eiu_ref, rep_u, hur_ref,
                        U_W1A, U_B1TAB, U_W2, BU_B2,
                        U_A1A, U_A1B, BU_BA1, U_A2, BU_BA2, U_A3M,
                        U_L1A, U_L1B, BU_BL1)

    neigh_s = attention_agg(esoc_ref[...], rep_u,
                            S_A1A, S_A1B, BS_BA1, S_A2, BS_BA2, S_A3M)
    social_space = relu(_mm(rep_u, ws(S_L1A)) + _mm(neigh_s, ws(S_L1B))
                        + bs(BS_BL1))

    i_lat = ui_agg(euu_ref, rep_i, hir_ref,
                   I_W1A, I_B1TAB, I_W2, BI_B2,
                   I_A1A, I_A1B, BI_BA1, I_A2, BI_BA2, I_A3M,
                   I_L1A, I_L1B, BI_BL1)

    u_lat = relu(_mm(item_space, ws(H_WUA)) + _mm(social_space, ws(H_WUB))
                 + bs(BH_BWU))
    u_lat = relu(_mm(u_lat, ws(H_WUR1)) + bs(BH_BUR1))
    u_lat = _mm(u_lat, ws(H_WUR2)) + bs(BH_BUR2)
    i_lat = relu(_mm(i_lat, ws(H_WIR1)) + bs(BH_BIR1))
    i_lat = _mm(i_lat, ws(H_WIR2)) + bs(BH_BIR2)
    lat = relu(_mm(u_lat, ws(H_WUI1A)) + _mm(i_lat, ws(H_WUI1B)) + bs(BH_BUI1))
    lat = relu(_mm(lat, ws(H_WUI2)) + bs(BH_BUI2))
    score = jnp.sum(lat * bs_ref[BH_W3][None, :], axis=-1)
    out_ref[...] = score


def _tc_forward(L, e_iu, e_uu, e_soc, rep_u, rep_i, hur, hir, WS, BS,
                interpret=False):
    B = rep_u.shape[0]
    nblk = B // BLK
    grid = (nblk,)
    body = functools.partial(_tc_body, L)
    out = pl.pallas_call(
        body,
        grid=grid,
        in_specs=[
            pl.BlockSpec((BLK * LP, D), lambda i: (i, 0)),
            pl.BlockSpec((BLK * LP, D), lambda i: (i, 0)),
            pl.BlockSpec((BLK * LP, D), lambda i: (i, 0)),
            pl.BlockSpec((BLK, D), lambda i: (i, 0)),
            pl.BlockSpec((BLK, D), lambda i: (i, 0)),
            pl.BlockSpec((BLK * LP, 1), lambda i: (i, 0)),
            pl.BlockSpec((BLK * LP, 1), lambda i: (i, 0)),
            pl.BlockSpec((NWS, D, D), lambda i: (0, 0, 0)),
            pl.BlockSpec((NBS, D), lambda i: (0, 0)),
        ],
        out_specs=pl.BlockSpec((BLK,), lambda i: (i,)),
        out_shape=jax.ShapeDtypeStruct((B,), jnp.float32),
        interpret=interpret,
    )(e_iu, e_uu, e_soc, rep_u, rep_i, hur, hir, WS, BS)
    return out


def _prep_weights(p):
    r2e = p['r2e']
    NR = r2e.shape[0]

    def tpose(l):
        return l['w'].T

    def pad_rows(m):
        return jnp.concatenate(
            [m, jnp.zeros((D - m.shape[0], D), jnp.float32)], axis=0)

    def a3mat(pa):
        return jnp.tile(pa['att3']['w'][0][:, None], (1, D))

    def agg_mats(pa):
        w1 = pa['w_r1']['w']
        b1tab = pad_rows(r2e @ w1[:, D:].T + pa['w_r1']['b'][None, :])
        return [w1[:, :D].T, b1tab, tpose(pa['w_r2']),
                pa['att1']['w'][:, :D].T, pa['att1']['w'][:, D:].T,
                tpose(pa['att2']), a3mat(pa),
                pa['linear1']['w'][:, :D].T, pa['linear1']['w'][:, D:].T]

    def agg_biases(pa):
        return [pa['w_r2']['b'], pa['att1']['b'], pa['att2']['b'],
                pa['att3']['w'][0], pa['linear1']['b']]

    def soc_mats(pa):
        return [pa['att1']['w'][:, :D].T, pa['att1']['w'][:, D:].T,
                tpose(pa['att2']), a3mat(pa),
                pa['linear1']['w'][:, :D].T, pa['linear1']['w'][:, D:].T]

    def soc_biases(pa):
        return [pa['att1']['b'], pa['att2']['b'],
                pa['att3']['w'][0], pa['linear1']['b']]

    g1, bb1 = p['bn1']['g'], p['bn1']['b']
    g2, bb2 = p['bn2']['g'], p['bn2']['b']
    g3, bb3 = p['bn3']['g'], p['bn3']['b']
    g4, bb4 = p['bn4']['g'], p['bn4']['b']

    wur1 = p['w_ur1']['w'].T * g1[None, :]
    bur1 = p['w_ur1']['b'] * g1 + bb1
    wir1 = p['w_ir1']['w'].T * g2[None, :]
    bir1 = p['w_ir1']['b'] * g2 + bb2
    wui1 = p['w_ui1']['w']
    wui1a = wui1[:, :D].T * g3[None, :]
    wui1b = wui1[:, D:].T * g3[None, :]
    bui1 = p['w_ui1']['b'] * g3 + bb3

    wui2 = p['w_ui2']['w'].T * g4[None, :]
    wui2 = jnp.concatenate(
        [wui2, jnp.zeros((D, D - 16), jnp.float32)], axis=1)
    bui2 = jnp.concatenate(
        [p['w_ui2']['b'] * g4 + bb4,
         jnp.ones((1,), jnp.float32),
         jnp.zeros((D - 17,), jnp.float32)])
    w3 = jnp.concatenate(
        [p['w_ui3']['w'][0], p['w_ui3']['b'],
         jnp.zeros((D - 17,), jnp.float32)])

    mats = (agg_mats(p['enc_u']) + agg_mats(p['enc_i']) + soc_mats(p['enc_s'])
            + [p['w_u']['w'][:, :D].T, p['w_u']['w'][:, D:].T,
               wur1, tpose(p['w_ur2']), wir1, tpose(p['w_ir2']),
               wui1a, wui1b, wui2])
    biases = (agg_biases(p['enc_u']) + agg_biases(p['enc_i'])
              + soc_biases(p['enc_s'])
              + [p['w_u']['b'], bur1, p['w_ur2']['b'], bir1, p['w_ir2']['b'],
                 bui1, bui2, w3]
              + [jnp.zeros((D,), jnp.float32)] * (NBS - 22))
    WS = jnp.stack(mats)
    BS = jnp.stack(biases)
    return WS, BS


def kernel(nodes_u, nodes_i, hist_u, hist_ur, hist_i, hist_ir, soc, params):
    p = params
    B, L = hist_u.shape
    S = soc.shape[1]
    NU = p['u2e'].shape[0]
    NI = p['i2e'].shape[0]

    WS, BS = _prep_weights(p)

    NCHUNK = 1
    Bc = B // NCHUNK
    sc_gather = _build_sc_gather(Bc, L, S, NU, NI)

    hu = hist_u.reshape(NCHUNK, Bc * L)
    hi = hist_i.reshape(NCHUNK, Bc * L)
    so = soc.reshape(NCHUNK, Bc * S)
    nu = nodes_u.reshape(NCHUNK, Bc)
    ni = nodes_i.reshape(NCHUNK, Bc)
    pad = ((0, 0), (0, LP - L))
    hur = jnp.pad(hist_ur, pad).reshape(NCHUNK, Bc * LP, 1)
    hir = jnp.pad(hist_ir, pad).reshape(NCHUNK, Bc * LP, 1)

    outs = []
    for c in range(NCHUNK):
        e_iu, e_uu, e_soc, rep_u, rep_i = sc_gather(
            p['u2e'], p['i2e'], hu[c], hi[c], so[c], nu[c], ni[c])
        outs.append(_tc_forward(L, e_iu, e_uu, e_soc, rep_u, rep_i,
                                hur[c], hir[c], WS, BS))
    return jnp.concatenate(outs)
